# baseline (speedup 1.0000x reference)
import jax
import jax.numpy as jnp
from jax.experimental import pallas as pl
from jax.experimental.pallas import tpu as pltpu

_CHUNKS = 8
_ROWS = 16 * 576
_CH_ROWS = _ROWS // _CHUNKS


def _copy_body(x_ref, o_ref, buf, in_sem, out_sem):
    ins = []
    for i in range(_CHUNKS):
        c = pltpu.make_async_copy(
            x_ref.at[pl.ds(i * _CH_ROWS, _CH_ROWS), :], buf.at[i], in_sem.at[i]
        )
        c.start()
        ins.append(c)
    outs = []
    for i in range(_CHUNKS):
        ins[i].wait()
        c = pltpu.make_async_copy(
            buf.at[i], o_ref.at[pl.ds(i * _CH_ROWS, _CH_ROWS), :], out_sem.at[i]
        )
        c.start()
        outs.append(c)
    for c in outs:
        c.wait()


def kernel(x):
    out = pl.pallas_call(
        _copy_body,
        in_specs=[pl.BlockSpec(memory_space=pl.ANY)],
        out_specs=pl.BlockSpec(memory_space=pl.ANY),
        out_shape=jax.ShapeDtypeStruct((_ROWS, 512), x.dtype),
        scratch_shapes=[
            pltpu.VMEM((_CHUNKS, _CH_ROWS, 512), x.dtype),
            pltpu.SemaphoreType.DMA((_CHUNKS,)),
            pltpu.SemaphoreType.DMA((_CHUNKS,)),
        ],
    )(x.reshape(_ROWS, 512))
    return out.reshape(x.shape)


# speedup vs baseline: 1.0386x; 1.0386x over previous
import jax
import jax.numpy as jnp
from jax.experimental import pallas as pl
from jax.experimental.pallas import tpu as pltpu

_CHUNKS = 2
_ROWS = 16 * 576
_CH_ROWS = _ROWS // _CHUNKS


def _copy_body(x_ref, o_ref, buf, in_sem, out_sem):
    ins = []
    for i in range(_CHUNKS):
        c = pltpu.make_async_copy(
            x_ref.at[pl.ds(i * _CH_ROWS, _CH_ROWS), :], buf.at[i], in_sem.at[i]
        )
        c.start()
        ins.append(c)
    outs = []
    for i in range(_CHUNKS):
        ins[i].wait()
        c = pltpu.make_async_copy(
            buf.at[i], o_ref.at[pl.ds(i * _CH_ROWS, _CH_ROWS), :], out_sem.at[i]
        )
        c.start()
        outs.append(c)
    for c in outs:
        c.wait()


def kernel(x):
    out = pl.pallas_call(
        _copy_body,
        in_specs=[pl.BlockSpec(memory_space=pl.ANY)],
        out_specs=pl.BlockSpec(memory_space=pl.ANY),
        out_shape=jax.ShapeDtypeStruct((_ROWS, 512), x.dtype),
        scratch_shapes=[
            pltpu.VMEM((_CHUNKS, _CH_ROWS, 512), x.dtype),
            pltpu.SemaphoreType.DMA((_CHUNKS,)),
            pltpu.SemaphoreType.DMA((_CHUNKS,)),
        ],
    )(x.reshape(_ROWS, 512))
    return out.reshape(x.shape)
